# Initial kernel scaffold; baseline (speedup 1.0000x reference)
import jax
import jax.numpy as jnp
from jax.experimental import pallas as pl


def kernel(node_features, edge_features, edge_to_edge_adj_matrix, node_to_edge_adj_matrix, weight_node, weight_edge, parameter_vector_edge, parameter_vector_node):
    raise NotImplementedError("write your pallas kernel here")


# speedup vs baseline: 1.6566x; 1.6566x over previous
import functools

import jax
import jax.numpy as jnp
from jax import lax
from jax.experimental import pallas as pl
from jax.experimental.pallas import tpu as pltpu

NEG = -1e30
BR = 256
BC = 512


def _leaky(x):
    return jnp.where(x >= 0, x, 0.01 * x)


def _online_step(part, base, adj, v, m_ref, l_ref, c_ref, acc_ref):
    a = _leaky(part + base)
    mask = adj > 0
    logit = jnp.where(mask, a, NEG)
    bmax = jnp.max(logit, axis=0, keepdims=True)
    m_new = jnp.maximum(m_ref[...], bmax)
    alpha = jnp.exp(m_ref[...] - m_new)
    p = jnp.exp(logit - m_new)
    m_ref[...] = m_new
    l_ref[...] = l_ref[...] * alpha + jnp.sum(p, axis=0, keepdims=True)
    c_ref[...] = c_ref[...] + jnp.sum(mask.astype(jnp.float32), axis=0, keepdims=True)
    acc_ref[...] = acc_ref[...] * alpha + lax.dot_general(
        v, p, (((0,), (0,)), ((), ())), preferred_element_type=jnp.float32)


def _attn_kernel(n2e_ref, e2e_ref, nf_ref, ef_ref, wn_ref, we_ref,
                 pvn_ref, pve_ref, out_n_ref, out_e_ref,
                 m_n, l_n, c_n, acc_n, m_e, l_e, c_e, acc_e,
                 *, nj, ee):
    i = pl.program_id(0)
    j = pl.program_id(1)

    @pl.when(j == 0)
    def _init():
        for m in (m_n, m_e):
            m[...] = jnp.full(m.shape, NEG, jnp.float32)
        for z in (l_n, c_n, acc_n, l_e, c_e, acc_e):
            z[...] = jnp.zeros(z.shape, jnp.float32)

    cd = (((1,), (1,)), ((), ()))
    pvn = pvn_ref[...]
    pve = pve_ref[...]
    wn_vec = lax.dot_general(wn_ref[...], pvn[:, ee:], cd,
                             preferred_element_type=jnp.float32)
    we_n = lax.dot_general(we_ref[...], pvn[:, :ee], cd,
                           preferred_element_type=jnp.float32)
    we_b = lax.dot_general(we_ref[...], pve[:, :ee], cd,
                           preferred_element_type=jnp.float32)
    we_p = lax.dot_general(we_ref[...], pve[:, ee:], cd,
                           preferred_element_type=jnp.float32)

    ef_i = ef_ref[pl.ds(i * BR, BR), :]
    c0 = (((0,), (1,)), ((), ()))
    base_n = lax.dot_general(we_n, ef_i, c0,
                             preferred_element_type=jnp.float32)
    base_e = lax.dot_general(we_b, ef_i, c0,
                             preferred_element_type=jnp.float32)

    nf_j = nf_ref[pl.ds(j * BC, BC), :]
    ef_j = ef_ref[pl.ds(j * BC, BC), :]
    part_n = jnp.dot(nf_j, wn_vec, preferred_element_type=jnp.float32)
    part_e = jnp.dot(ef_j, we_p, preferred_element_type=jnp.float32)

    _online_step(part_n, base_n, n2e_ref[...], nf_j, m_n, l_n, c_n, acc_n)
    _online_step(part_e, base_e, e2e_ref[...], ef_j, m_e, l_e, c_e, acc_e)

    @pl.when(j == nj - 1)
    def _finalize():
        c00 = (((0,), (0,)), ((), ()))
        means_n = acc_n[...] / (l_n[...] * c_n[...])
        o_n = lax.dot_general(wn_ref[...], means_n, c00,
                              preferred_element_type=jnp.float32)
        out_n_ref[...] = jnp.transpose(_leaky(o_n))
        means_e = acc_e[...] / (l_e[...] * c_e[...])
        o_e = lax.dot_general(we_ref[...], means_e, c00,
                              preferred_element_type=jnp.float32)
        out_e_ref[...] = jnp.transpose(_leaky(o_e))


def kernel(node_features, edge_features, edge_to_edge_adj_matrix,
           node_to_edge_adj_matrix, weight_node, weight_edge,
           parameter_vector_edge, parameter_vector_node):
    n, nf = node_features.shape
    e, ef = edge_features.shape
    ne = weight_node.shape[1]
    ee = weight_edge.shape[1]
    ni = e // BR
    nj = n // BC

    pvn = parameter_vector_node.reshape(1, -1)
    pve = parameter_vector_edge.reshape(1, -1)

    out_nodes, out_edges = pl.pallas_call(
        functools.partial(_attn_kernel, nj=nj, ee=ee),
        grid=(ni, nj),
        in_specs=[
            pl.BlockSpec((BC, BR), lambda i, j: (j, i)),
            pl.BlockSpec((BC, BR), lambda i, j: (j, i)),
            pl.BlockSpec((n, nf), lambda i, j: (0, 0)),
            pl.BlockSpec((e, ef), lambda i, j: (0, 0)),
            pl.BlockSpec((nf, ne), lambda i, j: (0, 0)),
            pl.BlockSpec((ef, ee), lambda i, j: (0, 0)),
            pl.BlockSpec((1, ee + ne), lambda i, j: (0, 0)),
            pl.BlockSpec((1, 2 * ee), lambda i, j: (0, 0)),
        ],
        out_specs=[
            pl.BlockSpec((BR, ne), lambda i, j: (i, 0)),
            pl.BlockSpec((BR, ee), lambda i, j: (i, 0)),
        ],
        out_shape=[
            jax.ShapeDtypeStruct((e, ne), jnp.float32),
            jax.ShapeDtypeStruct((e, ee), jnp.float32),
        ],
        scratch_shapes=[
            pltpu.VMEM((1, BR), jnp.float32),
            pltpu.VMEM((1, BR), jnp.float32),
            pltpu.VMEM((1, BR), jnp.float32),
            pltpu.VMEM((nf, BR), jnp.float32),
            pltpu.VMEM((1, BR), jnp.float32),
            pltpu.VMEM((1, BR), jnp.float32),
            pltpu.VMEM((1, BR), jnp.float32),
            pltpu.VMEM((ef, BR), jnp.float32),
        ],
    )(node_to_edge_adj_matrix, edge_to_edge_adj_matrix, node_features,
      edge_features, weight_node, weight_edge, pvn, pve)

    return jnp.concatenate([out_nodes, out_edges], axis=1)


# speedup vs baseline: 2.0142x; 1.2158x over previous
import functools

import jax
import jax.numpy as jnp
from jax import lax
from jax.experimental import pallas as pl
from jax.experimental.pallas import tpu as pltpu

NEG = -1e30
BR = 256
BC = 512


def _leaky(x):
    return jnp.maximum(x, 0.01 * x)


def _attn_kernel(n2e_ref, e2e_ref, nf_ref, ef_ref, wn_ref, we_ref,
                 pvn_ref, pve_ref, out_n_ref, out_e_ref,
                 l_n, c_n, acc_n, l_e, c_e, acc_e,
                 part_n, part_e, base_n, base_e, m_n, m_e, mp_n, mp_e,
                 *, nj, ee):
    i = pl.program_id(0)
    j = pl.program_id(1)
    cd = (((1,), (1,)), ((), ()))
    c0 = (((0,), (1,)), ((), ()))
    f32 = jnp.float32

    @pl.when((i == 0) & (j == 0))
    def _precompute():
        pvn = pvn_ref[...]
        pve = pve_ref[...]
        wn_vec = lax.dot_general(wn_ref[...], pvn[:, ee:], cd,
                                 preferred_element_type=f32)
        we_p = lax.dot_general(we_ref[...], pve[:, ee:], cd,
                               preferred_element_type=f32)
        pn = jnp.dot(nf_ref[...], wn_vec, preferred_element_type=f32)
        pe = jnp.dot(ef_ref[...], we_p, preferred_element_type=f32)
        part_n[...] = pn
        part_e[...] = pe
        mp_n[...] = jnp.max(pn, axis=0, keepdims=True)
        mp_e[...] = jnp.max(pe, axis=0, keepdims=True)

    @pl.when(j == 0)
    def _row_block_init():
        pvn = pvn_ref[...]
        pve = pve_ref[...]
        we_n = lax.dot_general(we_ref[...], pvn[:, :ee], cd,
                               preferred_element_type=f32)
        we_b = lax.dot_general(we_ref[...], pve[:, :ee], cd,
                               preferred_element_type=f32)
        ef_i = ef_ref[pl.ds(i * BR, BR), :]
        bn = lax.dot_general(we_n, ef_i, c0, preferred_element_type=f32)
        be = lax.dot_general(we_b, ef_i, c0, preferred_element_type=f32)
        base_n[...] = bn
        base_e[...] = be
        m_n[...] = _leaky(bn + mp_n[...])
        m_e[...] = _leaky(be + mp_e[...])
        for z in (l_n, c_n, acc_n, l_e, c_e, acc_e):
            z[...] = jnp.zeros(z.shape, f32)

    c00 = (((0,), (0,)), ((), ()))

    def _step(part, base, m, adj, v, l_ref, c_ref, acc_ref):
        a = _leaky(part + base)
        mask = adj > 0
        p = jnp.where(mask, jnp.exp(a - m), 0.0)
        l_ref[...] += jnp.sum(p, axis=0, keepdims=True)
        c_ref[...] += jnp.sum(mask.astype(f32), axis=0, keepdims=True)
        acc_ref[...] += lax.dot_general(v, p, c00, preferred_element_type=f32)

    _step(part_n[pl.ds(j * BC, BC), :], base_n[...], m_n[...], n2e_ref[...],
          nf_ref[pl.ds(j * BC, BC), :], l_n, c_n, acc_n)
    _step(part_e[pl.ds(j * BC, BC), :], base_e[...], m_e[...], e2e_ref[...],
          ef_ref[pl.ds(j * BC, BC), :], l_e, c_e, acc_e)

    @pl.when(j == nj - 1)
    def _finalize():
        means_n = acc_n[...] / (l_n[...] * c_n[...])
        o_n = lax.dot_general(wn_ref[...], means_n, c00,
                              preferred_element_type=f32)
        out_n_ref[...] = jnp.transpose(_leaky(o_n))
        means_e = acc_e[...] / (l_e[...] * c_e[...])
        o_e = lax.dot_general(we_ref[...], means_e, c00,
                              preferred_element_type=f32)
        out_e_ref[...] = jnp.transpose(_leaky(o_e))


def kernel(node_features, edge_features, edge_to_edge_adj_matrix,
           node_to_edge_adj_matrix, weight_node, weight_edge,
           parameter_vector_edge, parameter_vector_node):
    n, nf = node_features.shape
    e, ef = edge_features.shape
    ne = weight_node.shape[1]
    ee = weight_edge.shape[1]
    ni = e // BR
    nj = n // BC

    pvn = parameter_vector_node.reshape(1, -1)
    pve = parameter_vector_edge.reshape(1, -1)

    out_nodes, out_edges = pl.pallas_call(
        functools.partial(_attn_kernel, nj=nj, ee=ee),
        grid=(ni, nj),
        in_specs=[
            pl.BlockSpec((BC, BR), lambda i, j: (j, i)),
            pl.BlockSpec((BC, BR), lambda i, j: (j, i)),
            pl.BlockSpec((n, nf), lambda i, j: (0, 0)),
            pl.BlockSpec((e, ef), lambda i, j: (0, 0)),
            pl.BlockSpec((nf, ne), lambda i, j: (0, 0)),
            pl.BlockSpec((ef, ee), lambda i, j: (0, 0)),
            pl.BlockSpec((1, ee + ne), lambda i, j: (0, 0)),
            pl.BlockSpec((1, 2 * ee), lambda i, j: (0, 0)),
        ],
        out_specs=[
            pl.BlockSpec((BR, ne), lambda i, j: (i, 0)),
            pl.BlockSpec((BR, ee), lambda i, j: (i, 0)),
        ],
        out_shape=[
            jax.ShapeDtypeStruct((e, ne), jnp.float32),
            jax.ShapeDtypeStruct((e, ee), jnp.float32),
        ],
        scratch_shapes=[
            pltpu.VMEM((1, BR), jnp.float32),
            pltpu.VMEM((1, BR), jnp.float32),
            pltpu.VMEM((nf, BR), jnp.float32),
            pltpu.VMEM((1, BR), jnp.float32),
            pltpu.VMEM((1, BR), jnp.float32),
            pltpu.VMEM((ef, BR), jnp.float32),
            pltpu.VMEM((n, 1), jnp.float32),
            pltpu.VMEM((e, 1), jnp.float32),
            pltpu.VMEM((1, BR), jnp.float32),
            pltpu.VMEM((1, BR), jnp.float32),
            pltpu.VMEM((1, BR), jnp.float32),
            pltpu.VMEM((1, BR), jnp.float32),
            pltpu.VMEM((1, 1), jnp.float32),
            pltpu.VMEM((1, 1), jnp.float32),
        ],
    )(node_to_edge_adj_matrix, edge_to_edge_adj_matrix, node_features,
      edge_features, weight_node, weight_edge, pvn, pve)

    return jnp.concatenate([out_nodes, out_edges], axis=1)
